# baseline (speedup 1.0000x reference)
import functools
import math

import jax
import jax.numpy as jnp
import numpy as np
from jax import lax
from jax.experimental import pallas as pl
from jax.experimental.pallas import tpu as pltpu
from jax.experimental.pallas import tpu_sc as plsc

_CP, _CN = np.float32(0.9), np.float32(0.1)
_ALPHA = np.float32(0.25)
_ANCHORS_ALL = np.array(
    [[12, 16], [19, 36], [40, 28], [36, 75], [76, 55], [72, 146],
     [142, 110], [192, 243], [459, 401]], dtype=np.float32)
_GRIDS = (80, 40, 20)
_OBJ_NORM = (4.0, 1.0, 0.4)
_NT = 200
_NA = 3
_NE = 9 * _NA * _NT
_NEP = 6144
_ROWS2D = 48
_OFFX = (0.0, 0.5, 0.0, -0.5, 0.0, 0.5, -0.5, -0.5, 0.5)
_OFFY = (0.0, 0.0, 0.5, 0.0, -0.5, 0.5, 0.5, -0.5, 0.5)
_SC_NC, _SC_NS = 2, 16
_NTILES = _SC_NC * _SC_NS
_RPT = _NEP // _NTILES
_CONF_N = (153600 // _NTILES, 38400 // _NTILES, 9600 // _NTILES)
_CONF_P = (4800, 1200, 304)

_POLY_B = (2.6193049052380957e-05, -0.00013743391900788993, 9.621467324905097e-05, 0.0005957718822173774, 0.00010085963731398806, -0.004605502355843782, -0.005180348642170429, 0.028835618868470192, 0.10749127715826035, 0.15197207033634186, 0.08317766338586807)
_POLY_C = (-3.156950697302818e-05, 0.0001563110708957538, -6.418761040549725e-05, -0.0008242421317845583, -5.9911304560955614e-05, 0.0061352308839559555, 0.003454425372183323, -0.03844764456152916, -0.0716608390212059, -0.2026294320821762, -0.05545177310705185)
_POLY_D = (1.7378897609887645e-05, -8.515729859936982e-05, 2.965841122204438e-05, 0.00042621546890586615, 0.0002594321849755943, -0.003245654748752713, -0.00531292287632823, 0.02061331272125244, 0.09728311747312546, 0.1670406013727188, 0.12130075693130493)
_POLY_P = (-5.483997028932208e-06, 2.7636857339530252e-05, -3.4898127978522098e-06, -0.0002104660088662058, 7.562704240626772e-07, 0.0020828291308134794, 1.5193300839655421e-07, -0.02083335816860199, 1.963987861230976e-09, 0.25, 0.5)
_POLY_L = (1.875235611237258e-08, 6.155467872304143e-06, -3.515301796142012e-05, 7.566905878775287e-06, 0.00034307080204598606, 1.4684846973977983e-06, -0.00520866084843874, 4.363757710734717e-08, 0.125, -0.5, 0.6931471824645996)
_ATAN_C = (0.0011681264, -0.007568499, 0.023024166, -0.04519817, 0.06772865,
           -0.08822393, 0.11060458, -0.14280018, 0.19999667, -0.33333325, 1.0)


def _poly(coefs, x):
    p = jnp.full_like(x, np.float32(coefs[0]))
    for cc in coefs[1:]:
        p = p * x + np.float32(cc)
    return p


def _atan_pos(x):
    z = jnp.minimum(x, 1.0 / x)
    t = z * _poly(_ATAN_C, z * z)
    return jnp.where(x > 1.0, np.float32(math.pi / 2) - t, t)


def _frac(x):
    return x - jnp.floor(x)


def _sig(z):
    return 1.0 / (1.0 + jnp.exp(-z))


def _f_obj_var(c, p, lam, t):
    a_t = _ALPHA * t + (1.0 - _ALPHA) * (1.0 - t)
    om = p + t * (1.0 - 2.0 * p)
    return a_t * om * om * (c - c * t + lam)


def _entry_fields(of, af, vf, yt, G, head):
    Gf = np.float32(G)
    anchors = _ANCHORS_ALL[3 * head:3 * head + 3] / np.float32(640.0) * Gf
    b = yt[0].astype(jnp.int32)
    gx, gy = yt[2] * Gf, yt[3] * Gf
    gw, gh = yt[4] * Gf, yt[5] * Gf
    aw = jnp.where(af == 0.0, anchors[0, 0],
                   jnp.where(af == 1.0, anchors[1, 0], anchors[2, 0]))
    ah = jnp.where(af == 0.0, anchors[0, 1],
                   jnp.where(af == 1.0, anchors[1, 1], anchors[2, 1]))
    rw, rh = gw / aw, gh / ah
    rmax = jnp.maximum(jnp.maximum(rw, 1.0 / rw), jnp.maximum(rh, 1.0 / rh))
    keep = rmax < 4.0
    gxi_x, gxi_y = Gf - gx, Gf - gy
    fx, fy = _frac(gx), _frac(gy)
    fxi, fyi = _frac(gxi_x), _frac(gxi_y)
    j_ = (fx < 0.5) & (gx > 1.0)
    k_ = (fy < 0.5) & (gy > 1.0)
    l_ = (fxi < 0.5) & (gxi_x > 1.0)
    m_ = (fyi < 0.5) & (gxi_y > 1.0)
    js = (fx < 0.35) & (gx > 1.0)
    ks = (fy < 0.35) & (gy > 1.0)
    ls = (fxi < 0.35) & (gxi_x > 1.0)
    ms = (fyi < 0.35) & (gxi_y > 1.0)
    one = jnp.ones_like(of)
    zero = jnp.zeros_like(of)
    asf = lambda bb: jnp.where(bb, one, zero)
    rowm = (one, asf(j_), asf(k_), asf(l_), asf(m_), asf(js & ks),
            asf(ks & ls), asf(ls & ms), asf(ms & js))
    omask = rowm[8]
    offx = jnp.full_like(of, _OFFX[8])
    offy = jnp.full_like(of, _OFFY[8])
    for o in range(7, -1, -1):
        sel = of == np.float32(o)
        omask = jnp.where(sel, rowm[o], omask)
        offx = jnp.where(sel, np.float32(_OFFX[o]), offx)
        offy = jnp.where(sel, np.float32(_OFFY[o]), offy)
    gijx = (gx - offx).astype(jnp.int32)
    gijy = (gy - offy).astype(jnp.int32)
    gi = jnp.clip(gijx, 0, G - 1)
    gj = jnp.clip(gijy, 0, G - 1)
    a_i = af.astype(jnp.int32)
    row = ((b * 3 + a_i) * G + gj) * G + gi
    maskf = omask * jnp.where(keep, one, zero) * vf
    return dict(row=row, maskf=maskf,
                tbx=gx - gijx.astype(jnp.float32),
                tby=gy - gijy.astype(jnp.float32),
                tbw=gw, tbh=gh, aw=aw, ah=ah, tcls=yt[1])


def _prep_body(of_ref, af_ref, vf_ref, y0, y1, y2, y3, y4, y5, rows_ref):
    of, af, vf = of_ref[...], af_ref[...], vf_ref[...]
    yt = (y0[...], y1[...], y2[...], y3[...], y4[...], y5[...])
    for h, G in enumerate(_GRIDS):
        f = _entry_fields(of, af, vf, yt, G, h)
        rows_ref[h, :, :] = jnp.where(vf > 0.0, f["row"], 0)


def _prep(of, af, vf, ycols):
    return pl.pallas_call(
        _prep_body,
        out_shape=jax.ShapeDtypeStruct((3, _ROWS2D, 128), jnp.int32),
    )(of, af, vf, *ycols)


def _sc_body(tab0, tab1, tab2, rows,
             g0, g1, g2, cf0, cf1, cf2,
             idx2, rb0, rb1, rb2, semg, semc):
    cid = lax.axis_index("c")
    sid = lax.axis_index("s")
    wid = sid * _SC_NC + cid
    base = wid * _RPT
    half = _RPT // 2
    tabs = (tab0, tab1, tab2)
    rbufs = (rb0, rb1, rb2)
    gouts = (g0, g1, g2)
    cfs = (cf0, cf1, cf2)
    confcp = []
    for h in range(3):
        n = _CONF_N[h]
        confcp.append(pltpu.async_copy(
            tabs[h].at[pl.ds(wid * n, n), pl.ds(0, 8)],
            cfs[h].at[wid, pl.ds(0, n)], semc))
    gathers = []
    for h in range(3):
        hoff = h * _NEP + base
        pltpu.sync_copy(rows.at[pl.ds(hoff, half)], idx2.at[2 * h])
        pltpu.sync_copy(rows.at[pl.ds(hoff + half, half)], idx2.at[2 * h + 1])
        gathers.append(pltpu.async_copy(
            tabs[h].at[idx2.at[2 * h]], rbufs[h].at[pl.ds(0, half)], semg))
        gathers.append(pltpu.async_copy(
            tabs[h].at[idx2.at[2 * h + 1]], rbufs[h].at[pl.ds(half, half)], semg))
    for g in gathers:
        g.wait()
    for h in range(3):
        pltpu.sync_copy(rbufs[h], gouts[h].at[pl.ds(base, _RPT)])
    for c in confcp:
        c.wait()


def _sc_run(tabs, rows):
    mesh = plsc.VectorSubcoreMesh(core_axis_name="c", subcore_axis_name="s")
    out_type = (
        jax.ShapeDtypeStruct((_NEP, 85), jnp.float32),
        jax.ShapeDtypeStruct((_NEP, 85), jnp.float32),
        jax.ShapeDtypeStruct((_NEP, 85), jnp.float32),
        jax.ShapeDtypeStruct((_NTILES, _CONF_P[0], 8), jnp.float32),
        jax.ShapeDtypeStruct((_NTILES, _CONF_P[1], 8), jnp.float32),
        jax.ShapeDtypeStruct((_NTILES, _CONF_P[2], 8), jnp.float32),
    )
    f = pl.kernel(
        _sc_body,
        out_type=out_type,
        mesh=mesh,
        compiler_params=pltpu.CompilerParams(use_tc_tiling_on_sc=False),
        scratch_types=[
            pltpu.VMEM((6, _RPT // 2), jnp.int32),
            pltpu.VMEM((_RPT, 85), jnp.float32),
            pltpu.VMEM((_RPT, 85), jnp.float32),
            pltpu.VMEM((_RPT, 85), jnp.float32),
            pltpu.SemaphoreType.DMA,
            pltpu.SemaphoreType.DMA,
        ],
    )
    return f(tabs[0], tabs[1], tabs[2], rows)


def _sparse_body(G, head, gat_ref, of_ref, af_ref, vf_ref,
                 y0, y1, y2, y3, y4, y5,
                 sbox_ref, scls_ref, sobj_ref, nv_ref):
    of, af, vf = of_ref[...], af_ref[...], vf_ref[...]
    yt = (y0[...], y1[...], y2[...], y3[...], y4[...], y5[...])
    f = _entry_fields(of, af, vf, yt, G, head)
    maskf = f["maskf"]
    gat = gat_ref[...]

    def col(k):
        return jnp.sum(gat[:, :, k:k + 1], axis=2)

    ps0, ps1, ps2, ps3, ps4 = col(0), col(1), col(2), col(3), col(4)
    pxy_x = _sig(ps1) * 2.0 - 0.5
    pxy_y = _sig(ps2) * 2.0 - 0.5
    tw2 = _sig(ps3) * 2.0
    th2 = _sig(ps4) * 2.0
    pw = tw2 * tw2 * f["aw"]
    ph = th2 * th2 * f["ah"]
    eps = np.float32(1e-7)
    x1c, y1c, w1, h1 = pxy_x, pxy_y, pw, ph
    x2c, y2c, w2, h2 = f["tbx"], f["tby"], f["tbw"], f["tbh"]
    b1x1, b1x2 = x1c - w1 * 0.5, x1c + w1 * 0.5
    b1y1, b1y2 = y1c - h1 * 0.5, y1c + h1 * 0.5
    b2x1, b2x2 = x2c - w2 * 0.5, x2c + w2 * 0.5
    b2y1, b2y2 = y2c - h2 * 0.5, y2c + h2 * 0.5
    iw = jnp.clip(jnp.minimum(b1x2, b2x2) - jnp.maximum(b1x1, b2x1), 0.0)
    ih = jnp.clip(jnp.minimum(b1y2, b2y2) - jnp.maximum(b1y1, b2y1), 0.0)
    inter = iw * ih
    union = w1 * h1 + w2 * h2 - inter + eps
    iou = inter / union
    cw = jnp.maximum(b1x2, b2x2) - jnp.minimum(b1x1, b2x1)
    ch = jnp.maximum(b1y2, b2y2) - jnp.minimum(b1y1, b2y1)
    c2 = cw * cw + ch * ch + eps
    rho2 = ((b2x1 + b2x2 - b1x1 - b1x2) ** 2 +
            (b2y1 + b2y2 - b1y1 - b1y2) ** 2) * 0.25
    datan = _atan_pos(w2 / (h2 + eps)) - _atan_pos(w1 / (h1 + eps))
    v = np.float32(4.0 / math.pi ** 2) * datan * datan
    alpha_c = v / (v - iou + np.float32(1.0 + 1e-7))
    ciou = iou - (rho2 / c2 + v * alpha_c)
    sbox_ref[...] = jnp.sum(maskf * (1.0 - ciou)).reshape(1, 1)
    c_e = _sig(ps0)
    p_e = _poly(_POLY_P, c_e)
    l_e = _poly(_POLY_L, c_e)
    v_t = _CP - 1.0 + jnp.clip(ciou, 0.0)
    s_obj = jnp.sum(maskf * (_f_obj_var(c_e, p_e, l_e, v_t) -
                             _f_obj_var(c_e, p_e, l_e, _CN)))
    sobj_ref[...] = s_obj.reshape(1, 1)
    q = _sig(gat[:, :, 5:])
    base_sum = jnp.sum(_poly(_POLY_B, q), axis=2)
    cls_iota = lax.broadcasted_iota(jnp.int32, q.shape, 2)
    onehot = cls_iota == f["tcls"].astype(jnp.int32)[:, :, None]
    q_t = jnp.sum(jnp.where(onehot, q, 0.0), axis=2)
    corr = _poly(_POLY_C, q_t)
    scls_ref[...] = jnp.sum(maskf * (base_sum + corr)).reshape(1, 1)
    nv_ref[...] = jnp.sum(maskf).reshape(1, 1)


def _sparse_head(head, gat3, of, af, vf, ycols):
    G = _GRIDS[head]
    scalar = jax.ShapeDtypeStruct((1, 1), jnp.float32)
    return pl.pallas_call(
        functools.partial(_sparse_body, G, head),
        out_shape=[scalar] * 4,
    )(gat3, of, af, vf, *ycols)


def _dense_body(c0_ref, c1_ref, c2_ref, out_ref):
    total = jnp.float32(0.0)
    for h, ref in enumerate((c0_ref, c1_ref, c2_ref)):
        raw = ref[...]
        c = _sig(raw)
        term = _poly(_POLY_D, c)
        sh = ref.shape
        lane = lax.broadcasted_iota(jnp.int32, sh, 1)
        keep = lane % 8 == 0
        if h == 2:
            flat = (lax.broadcasted_iota(jnp.int32, sh, 0).astype(jnp.float32) * 128.0
                    + lax.broadcasted_iota(jnp.int32, sh, 1).astype(jnp.float32))
            t = flat - 2432.0 * jnp.floor(flat / 2432.0)
            keep = keep & (t < 2400.0)
        term = jnp.where(keep, term, 0.0)
        G = _GRIDS[h]
        w = np.float32(_OBJ_NORM[h] / (8 * _NA * G * G))
        total = total + jnp.sum(term) * w
    out_ref[...] = total.reshape(1, 1)


def _dense(cf0, cf1, cf2):
    return pl.pallas_call(
        _dense_body,
        out_shape=jax.ShapeDtypeStruct((1, 1), jnp.float32),
    )(cf0.reshape(-1, 128), cf1.reshape(-1, 128), cf2.reshape(-1, 128))


def kernel(x0, x1, x2, y):
    xs = (x0, x1, x2)
    bs = x0.shape[0]
    e = np.arange(_NEP)
    of = jnp.asarray((np.minimum(e, _NE - 1) // (_NA * _NT)).astype(np.float32).reshape(_ROWS2D, 128))
    af = jnp.asarray(((np.minimum(e, _NE - 1) // _NT) % _NA).astype(np.float32).reshape(_ROWS2D, 128))
    vf = jnp.asarray((e < _NE).astype(np.float32).reshape(_ROWS2D, 128))
    ycols = []
    for k in range(6):
        colk = jnp.tile(y[:, k], 9 * _NA)
        colk = jnp.concatenate([colk, jnp.zeros((_NEP - _NE,), jnp.float32)])
        ycols.append(colk.reshape(_ROWS2D, 128))
    rows3 = _prep(of, af, vf, ycols)
    tabs = tuple(x.reshape(-1, 85) for x in xs)
    g0, g1, g2, cf0, cf1, cf2 = _sc_run(tabs, rows3.reshape(3 * _NEP))
    gs = (g0, g1, g2)
    dense = _dense(cf0, cf1, cf2)
    lbox = jnp.float32(0.0)
    lcls = jnp.float32(0.0)
    lobj = dense[0, 0]
    for h in range(3):
        G = _GRIDS[h]
        gat3 = gs[h].reshape(_ROWS2D, 128, 85)
        sbox, scls, sobj, nv = _sparse_head(h, gat3, of, af, vf, ycols)
        nvs = nv[0, 0]
        lbox = lbox + sbox[0, 0] / nvs
        lcls = lcls + scls[0, 0] / (nvs * 80.0)
        cells = np.float32(bs * _NA * G * G)
        lobj = lobj + sobj[0, 0] / cells * np.float32(_OBJ_NORM[h])
    loss = (lcls * np.float32(0.5) + lbox * np.float32(0.05) + lobj) * np.float32(bs)
    return jnp.reshape(loss, (1,))


# speedup vs baseline: 2.2979x; 2.2979x over previous
import functools
import math

import jax
import jax.numpy as jnp
import numpy as np
from jax import lax
from jax.experimental import pallas as pl
from jax.experimental.pallas import tpu as pltpu
from jax.experimental.pallas import tpu_sc as plsc

_CP, _CN = np.float32(0.9), np.float32(0.1)
_ALPHA = np.float32(0.25)
_ANCHORS_ALL = np.array(
    [[12, 16], [19, 36], [40, 28], [36, 75], [76, 55], [72, 146],
     [142, 110], [192, 243], [459, 401]], dtype=np.float32)
_GRIDS = (80, 40, 20)
_OBJ_NORM = (4.0, 1.0, 0.4)
_NT = 200
_NA = 3
_NE = 9 * _NA * _NT
_NEP = 6144
_ROWS2D = 48
_CHUNKS_B = 4
_ROWS_B = _ROWS2D // _CHUNKS_B
_OFFX = (0.0, 0.5, 0.0, -0.5, 0.0, 0.5, -0.5, -0.5, 0.5)
_OFFY = (0.0, 0.0, 0.5, 0.0, -0.5, 0.5, 0.5, -0.5, 0.5)
_SC_NC, _SC_NS = 2, 16
_NTILES = _SC_NC * _SC_NS
_RPT = _NEP // _NTILES

_ATAN_C = (0.0011681264, -0.007568499, 0.023024166, -0.04519817, 0.06772865,
           -0.08822393, 0.11060458, -0.14280018, 0.19999667, -0.33333325, 1.0)


def _atan_pos(x):
    z = jnp.minimum(x, 1.0 / x)
    x2 = z * z
    p = jnp.full_like(x, np.float32(_ATAN_C[0]))
    for cc in _ATAN_C[1:]:
        p = p * x2 + np.float32(cc)
    t = z * p
    return jnp.where(x > 1.0, np.float32(math.pi / 2) - t, t)


def _frac(x):
    return x - jnp.floor(x)


def _entry_fields(of, af, vf, yt, G, head):
    Gf = np.float32(G)
    anchors = _ANCHORS_ALL[3 * head:3 * head + 3] / np.float32(640.0) * Gf
    b = yt[0].astype(jnp.int32)
    gx, gy = yt[2] * Gf, yt[3] * Gf
    gw, gh = yt[4] * Gf, yt[5] * Gf
    aw = jnp.where(af == 0.0, anchors[0, 0],
                   jnp.where(af == 1.0, anchors[1, 0], anchors[2, 0]))
    ah = jnp.where(af == 0.0, anchors[0, 1],
                   jnp.where(af == 1.0, anchors[1, 1], anchors[2, 1]))
    rw, rh = gw / aw, gh / ah
    rmax = jnp.maximum(jnp.maximum(rw, 1.0 / rw), jnp.maximum(rh, 1.0 / rh))
    keep = rmax < 4.0
    gxi_x, gxi_y = Gf - gx, Gf - gy
    fx, fy = _frac(gx), _frac(gy)
    fxi, fyi = _frac(gxi_x), _frac(gxi_y)
    j_ = (fx < 0.5) & (gx > 1.0)
    k_ = (fy < 0.5) & (gy > 1.0)
    l_ = (fxi < 0.5) & (gxi_x > 1.0)
    m_ = (fyi < 0.5) & (gxi_y > 1.0)
    js = (fx < 0.35) & (gx > 1.0)
    ks = (fy < 0.35) & (gy > 1.0)
    ls = (fxi < 0.35) & (gxi_x > 1.0)
    ms = (fyi < 0.35) & (gxi_y > 1.0)
    one = jnp.ones_like(of)
    zero = jnp.zeros_like(of)
    asf = lambda bb: jnp.where(bb, one, zero)
    rowm = (one, asf(j_), asf(k_), asf(l_), asf(m_), asf(js & ks),
            asf(ks & ls), asf(ls & ms), asf(ms & js))
    omask = rowm[8]
    offx = jnp.full_like(of, _OFFX[8])
    offy = jnp.full_like(of, _OFFY[8])
    for o in range(7, -1, -1):
        sel = of == np.float32(o)
        omask = jnp.where(sel, rowm[o], omask)
        offx = jnp.where(sel, np.float32(_OFFX[o]), offx)
        offy = jnp.where(sel, np.float32(_OFFY[o]), offy)
    gijx = (gx - offx).astype(jnp.int32)
    gijy = (gy - offy).astype(jnp.int32)
    gi = jnp.clip(gijx, 0, G - 1)
    gj = jnp.clip(gijy, 0, G - 1)
    a_i = af.astype(jnp.int32)
    row = ((b * 3 + a_i) * G + gj) * G + gi
    maskf = omask * jnp.where(keep, one, zero) * vf
    return dict(row=row, maskf=maskf,
                tbx=gx - gijx.astype(jnp.float32),
                tby=gy - gijy.astype(jnp.float32),
                tbw=gw, tbh=gh, aw=aw, ah=ah, tcls=yt[1])


def _prep_body(of_ref, af_ref, vf_ref, y0, y1, y2, y3, y4, y5, rows_ref):
    of, af, vf = of_ref[...], af_ref[...], vf_ref[...]
    yt = (y0[...], y1[...], y2[...], y3[...], y4[...], y5[...])
    for h, G in enumerate(_GRIDS):
        f = _entry_fields(of, af, vf, yt, G, h)
        rows_ref[h, :, :] = jnp.where(vf > 0.0, f["row"], 0)


def _prep(of, af, vf, ycols):
    return pl.pallas_call(
        _prep_body,
        out_shape=jax.ShapeDtypeStruct((3, _ROWS2D, 128), jnp.int32),
    )(of, af, vf, *ycols)


def _sc_gather_body(tab0, tab1, tab2, rows, g0, g1, g2, idx2, rbuf, sem):
    cid = lax.axis_index("c")
    sid = lax.axis_index("s")
    wid = sid * _SC_NC + cid
    base = wid * _RPT
    half = _RPT // 2
    for h, (tab, gout) in enumerate(((tab0, g0), (tab1, g1), (tab2, g2))):
        hoff = h * _NEP + base
        pltpu.sync_copy(rows.at[pl.ds(hoff, half)], idx2.at[0])
        pltpu.sync_copy(rows.at[pl.ds(hoff + half, half)], idx2.at[1])
        cp0 = pltpu.async_copy(tab.at[idx2.at[0]], rbuf.at[pl.ds(0, half)], sem)
        cp1 = pltpu.async_copy(tab.at[idx2.at[1]], rbuf.at[pl.ds(half, half)], sem)
        cp0.wait()
        cp1.wait()
        pltpu.sync_copy(rbuf, gout.at[pl.ds(base, _RPT)])


def _sc_gather(tabs, rows):
    mesh = plsc.VectorSubcoreMesh(core_axis_name="c", subcore_axis_name="s")
    out_type = tuple(jax.ShapeDtypeStruct((_NEP, 85), jnp.float32) for _ in range(3))
    f = pl.kernel(
        _sc_gather_body,
        out_type=out_type,
        mesh=mesh,
        compiler_params=pltpu.CompilerParams(use_tc_tiling_on_sc=False),
        scratch_types=[
            pltpu.VMEM((2, _RPT // 2), jnp.int32),
            pltpu.VMEM((_RPT, 85), jnp.float32),
            pltpu.SemaphoreType.DMA,
        ],
    )
    return f(tabs[0], tabs[1], tabs[2], rows)


def _f_obj(c, t):
    p = 1.0 / (1.0 + jnp.exp(-c))
    p_t = t * p + (1.0 - t) * (1.0 - p)
    a_t = _ALPHA * t + (1.0 - _ALPHA) * (1.0 - t)
    om = 1.0 - p_t
    return a_t * om * om * (c - c * t + jnp.log1p(jnp.exp(-c)))


def _f_cls(pc, sig_pc, t, alpha):
    g = t - sig_pc
    bce = jnp.maximum(pc, 0.0) - pc * t + jnp.log1p(jnp.exp(-jnp.abs(pc)))
    return alpha * g * g * bce


def _sparse_body(G, head, gat_ref, of_ref, af_ref, vf_ref,
                 y0, y1, y2, y3, y4, y5,
                 sbox_ref, scls_ref, sobj_ref, nv_ref):
    c = pl.program_id(0)
    of, af, vf = of_ref[0], af_ref[0], vf_ref[0]
    yt = (y0[0], y1[0], y2[0], y3[0], y4[0], y5[0])
    f = _entry_fields(of, af, vf, yt, G, head)
    maskf = f["maskf"]
    gat = gat_ref[...]

    def col(k):
        return jnp.sum(gat[:, :, k:k + 1], axis=2)

    ps0, ps1, ps2, ps3, ps4 = col(0), col(1), col(2), col(3), col(4)
    sig = lambda z: 1.0 / (1.0 + jnp.exp(-z))
    pxy_x = sig(ps1) * 2.0 - 0.5
    pxy_y = sig(ps2) * 2.0 - 0.5
    tw2 = sig(ps3) * 2.0
    th2 = sig(ps4) * 2.0
    pw = tw2 * tw2 * f["aw"]
    ph = th2 * th2 * f["ah"]
    eps = np.float32(1e-7)
    x1c, y1c, w1, h1 = pxy_x, pxy_y, pw, ph
    x2c, y2c, w2, h2 = f["tbx"], f["tby"], f["tbw"], f["tbh"]
    b1x1, b1x2 = x1c - w1 * 0.5, x1c + w1 * 0.5
    b1y1, b1y2 = y1c - h1 * 0.5, y1c + h1 * 0.5
    b2x1, b2x2 = x2c - w2 * 0.5, x2c + w2 * 0.5
    b2y1, b2y2 = y2c - h2 * 0.5, y2c + h2 * 0.5
    iw = jnp.clip(jnp.minimum(b1x2, b2x2) - jnp.maximum(b1x1, b2x1), 0.0)
    ih = jnp.clip(jnp.minimum(b1y2, b2y2) - jnp.maximum(b1y1, b2y1), 0.0)
    inter = iw * ih
    union = w1 * h1 + w2 * h2 - inter + eps
    iou = inter / union
    cw = jnp.maximum(b1x2, b2x2) - jnp.minimum(b1x1, b2x1)
    ch = jnp.maximum(b1y2, b2y2) - jnp.minimum(b1y1, b2y1)
    c2 = cw * cw + ch * ch + eps
    rho2 = ((b2x1 + b2x2 - b1x1 - b1x2) ** 2 +
            (b2y1 + b2y2 - b1y1 - b1y2) ** 2) * 0.25
    datan = _atan_pos(w2 / (h2 + eps)) - _atan_pos(w1 / (h1 + eps))
    v = np.float32(4.0 / math.pi ** 2) * datan * datan
    alpha_c = v / (v - iou + np.float32(1.0 + 1e-7))
    ciou = iou - (rho2 / c2 + v * alpha_c)
    s_box = jnp.sum(maskf * (1.0 - ciou))
    c_e = sig(ps0)
    v_t = _CP - 1.0 + jnp.clip(ciou, 0.0)
    s_obj = jnp.sum(maskf * (_f_obj(c_e, v_t) - _f_obj(c_e, _CN)))
    pcls = sig(gat[:, :, 5:])
    sig_pc = sig(pcls)
    base = _f_cls(pcls, sig_pc, _CN, 1.0 - _ALPHA)
    corr = _f_cls(pcls, sig_pc, _CP, _ALPHA) - base
    cls_iota = lax.broadcasted_iota(jnp.int32, pcls.shape, 2)
    onehot = cls_iota == f["tcls"].astype(jnp.int32)[:, :, None]
    terms = base + jnp.where(onehot, corr, 0.0)
    s_cls = jnp.sum(maskf[:, :, None] * terms)
    s_nv = jnp.sum(maskf)

    @pl.when(c == 0)
    def _init():
        sbox_ref[...] = jnp.zeros_like(sbox_ref)
        scls_ref[...] = jnp.zeros_like(scls_ref)
        sobj_ref[...] = jnp.zeros_like(sobj_ref)
        nv_ref[...] = jnp.zeros_like(nv_ref)

    sbox_ref[...] += s_box
    scls_ref[...] += s_cls
    sobj_ref[...] += s_obj
    nv_ref[...] += s_nv


def _sparse_head(head, gat3, of, af, vf, ycols):
    G = _GRIDS[head]
    scalar = jax.ShapeDtypeStruct((1, 1), jnp.float32)
    in_specs = [pl.BlockSpec((_ROWS_B, 128, 85), lambda c: (c, 0, 0))]
    in_specs += [pl.BlockSpec((1, _ROWS_B, 128), lambda c: (c, 0, 0))] * 9
    out_specs = [pl.BlockSpec((1, 1), lambda c: (0, 0))] * 4
    aux4 = tuple(a.reshape(_CHUNKS_B, _ROWS_B, 128) for a in (of, af, vf, *ycols))
    return pl.pallas_call(
        functools.partial(_sparse_body, G, head),
        grid=(_CHUNKS_B,),
        in_specs=in_specs,
        out_specs=out_specs,
        out_shape=[scalar] * 4,
    )(gat3, *aux4)


def _dense_body(x_ref, out_ref):
    g = pl.program_id(0)
    col0 = jnp.sum(x_ref[:, :, 0:1], axis=2)
    c = 1.0 / (1.0 + jnp.exp(-col0))
    s = jnp.sum(_f_obj(c, _CN))

    @pl.when(g == 0)
    def _init():
        out_ref[...] = jnp.zeros_like(out_ref)

    out_ref[...] += s


def _dense_head(x3):
    nrow = x3.shape[0]
    blk = 75
    steps = nrow // blk
    return pl.pallas_call(
        _dense_body,
        grid=(steps,),
        in_specs=[pl.BlockSpec((blk, 128, 85), lambda g: (g, 0, 0))],
        out_specs=pl.BlockSpec((1, 1), lambda g: (0, 0)),
        out_shape=jax.ShapeDtypeStruct((1, 1), jnp.float32),
    )(x3)


def kernel(x0, x1, x2, y):
    xs = (x0, x1, x2)
    bs = x0.shape[0]
    e = np.arange(_NEP)
    of = jnp.asarray((np.minimum(e, _NE - 1) // (_NA * _NT)).astype(np.float32).reshape(_ROWS2D, 128))
    af = jnp.asarray(((np.minimum(e, _NE - 1) // _NT) % _NA).astype(np.float32).reshape(_ROWS2D, 128))
    vf = jnp.asarray((e < _NE).astype(np.float32).reshape(_ROWS2D, 128))
    ycols = []
    for k in range(6):
        colk = jnp.tile(y[:, k], 9 * _NA)
        colk = jnp.concatenate([colk, jnp.zeros((_NEP - _NE,), jnp.float32)])
        ycols.append(colk.reshape(_ROWS2D, 128))
    rows3 = _prep(of, af, vf, ycols)
    tabs = tuple(x.reshape(-1, 85) for x in xs)
    g0, g1, g2 = _sc_gather(tabs, rows3.reshape(3 * _NEP))
    gs = (g0, g1, g2)
    lbox = jnp.float32(0.0)
    lcls = jnp.float32(0.0)
    lobj = jnp.float32(0.0)
    for h in range(3):
        G = _GRIDS[h]
        gat3 = gs[h].reshape(_ROWS2D, 128, 85)
        sbox, scls, sobj, nv = _sparse_head(h, gat3, of, af, vf, ycols)
        dense = _dense_head(xs[h].reshape(-1, 128, 85))
        nvs = nv[0, 0]
        lbox = lbox + sbox[0, 0] / nvs
        lcls = lcls + scls[0, 0] / (nvs * 80.0)
        cells = np.float32(bs * _NA * G * G)
        lobj = lobj + (dense[0, 0] + sobj[0, 0]) / cells * np.float32(_OBJ_NORM[h])
    loss = (lcls * np.float32(0.5) + lbox * np.float32(0.05) + lobj) * np.float32(bs)
    return jnp.reshape(loss, (1,))
